# Initial kernel scaffold; baseline (speedup 1.0000x reference)
import jax
import jax.numpy as jnp
from jax.experimental import pallas as pl


def kernel(input, emb_table, conv_w, conv_b, w_proj, b_proj, w_gate, b_gate):
    raise NotImplementedError("write your pallas kernel here")


# speedup vs baseline: 1.9729x; 1.9729x over previous
import jax
import jax.numpy as jnp
from jax.experimental import pallas as pl
from jax.experimental.pallas import tpu as pltpu

E_CHAR = 50
EMBED = 256
CHAR_VOCAB = 96
KSIZE = 5
L = 21
T_OUT = L - KSIZE + 1

BLOCK = 512


def _block_body(ids_ref, emb_ref, convw_ref, convb_ref, wp_ref, bp_ref,
                wg_ref, bg_ref, out_ref):
    nb = ids_ref.shape[0]
    ids = ids_ref[...]
    iota_v = jax.lax.broadcasted_iota(jnp.int32, (nb, L, CHAR_VOCAB), 2)
    oh = (ids[:, :, None] == iota_v).astype(jnp.float32)
    oh2 = oh.reshape(nb * L, CHAR_VOCAB)
    e = jnp.dot(oh2, emb_ref[...], preferred_element_type=jnp.float32)
    e = e.reshape(nb, L, E_CHAR)
    acc = jnp.zeros((nb * T_OUT, EMBED), jnp.float32)
    for k in range(KSIZE):
        ek = e[:, k:k + T_OUT, :].reshape(nb * T_OUT, E_CHAR)
        acc = acc + jnp.dot(ek, convw_ref[k], preferred_element_type=jnp.float32)
    acc = acc + convb_ref[...][None, :]
    acc = jnp.maximum(acc, 0.0).reshape(nb, T_OUT, EMBED)
    xc = jnp.max(acc, axis=1)
    proj = jnp.maximum(
        jnp.dot(xc, wp_ref[...], preferred_element_type=jnp.float32)
        + bp_ref[...][None, :], 0.0)
    gate = jax.nn.sigmoid(
        jnp.dot(xc, wg_ref[...], preferred_element_type=jnp.float32)
        + bg_ref[...][None, :])
    out_ref[...] = gate * proj + (1.0 - gate) * xc


def kernel(input, emb_table, conv_w, conv_b, w_proj, b_proj, w_gate, b_gate):
    s, b, l = input.shape
    n = s * b
    ids = input.reshape(n, l).astype(jnp.int32)
    convw_t = jnp.transpose(conv_w, (2, 1, 0))
    wp_t = w_proj.T
    wg_t = w_gate.T

    grid = (n // BLOCK,)
    rep = lambda i: (0, 0)
    rep3 = lambda i: (0, 0, 0)
    out = pl.pallas_call(
        _block_body,
        grid=grid,
        in_specs=[
            pl.BlockSpec((BLOCK, l), lambda i: (i, 0)),
            pl.BlockSpec((CHAR_VOCAB, E_CHAR), rep),
            pl.BlockSpec((KSIZE, E_CHAR, EMBED), rep3),
            pl.BlockSpec((EMBED,), lambda i: (0,)),
            pl.BlockSpec((EMBED, EMBED), rep),
            pl.BlockSpec((EMBED,), lambda i: (0,)),
            pl.BlockSpec((EMBED, EMBED), rep),
            pl.BlockSpec((EMBED,), lambda i: (0,)),
        ],
        out_specs=pl.BlockSpec((BLOCK, EMBED), lambda i: (i, 0)),
        out_shape=jax.ShapeDtypeStruct((n, EMBED), jnp.float32),
    )(ids, emb_table, convw_t, conv_b, wp_t, b_proj, wg_t, b_gate)
    return out.reshape(s, b, EMBED)


# speedup vs baseline: 2.1943x; 1.1122x over previous
import jax
import jax.numpy as jnp
from jax.experimental import pallas as pl
from jax.experimental.pallas import tpu as pltpu

E_CHAR = 50
EMBED = 256
CHAR_VOCAB = 96
KSIZE = 5
L = 21
T_OUT = L - KSIZE + 1

BLOCK = 512


def _block_body(ids_ref, emb_ref, convw_ref, convb_ref, wp_ref, bp_ref,
                wg_ref, bg_ref, out_ref):
    nb = ids_ref.shape[0]
    ids = ids_ref[...]
    iota_v = jax.lax.broadcasted_iota(jnp.int32, (nb, L, CHAR_VOCAB), 2)
    oh = (ids[:, :, None] == iota_v).astype(jnp.bfloat16)
    oh2 = oh.reshape(nb * L, CHAR_VOCAB)
    e = jnp.dot(oh2, emb_ref[...].astype(jnp.bfloat16),
                preferred_element_type=jnp.float32)
    e = e.astype(jnp.bfloat16).reshape(nb, L, E_CHAR)
    acc = jnp.zeros((nb * T_OUT, EMBED), jnp.float32)
    for k in range(KSIZE):
        ek = e[:, k:k + T_OUT, :].reshape(nb * T_OUT, E_CHAR)
        acc = acc + jnp.dot(ek, convw_ref[k].astype(jnp.bfloat16),
                            preferred_element_type=jnp.float32)
    acc = acc + convb_ref[...][None, :]
    acc = jnp.maximum(acc, 0.0).reshape(nb, T_OUT, EMBED)
    xc = jnp.max(acc, axis=1)
    proj = jnp.maximum(
        jnp.dot(xc, wp_ref[...], preferred_element_type=jnp.float32)
        + bp_ref[...][None, :], 0.0)
    gate = jax.nn.sigmoid(
        jnp.dot(xc, wg_ref[...], preferred_element_type=jnp.float32)
        + bg_ref[...][None, :])
    out_ref[...] = gate * proj + (1.0 - gate) * xc


def kernel(input, emb_table, conv_w, conv_b, w_proj, b_proj, w_gate, b_gate):
    s, b, l = input.shape
    n = s * b
    ids = input.reshape(n, l).astype(jnp.int32)
    convw_t = jnp.transpose(conv_w, (2, 1, 0))
    wp_t = w_proj.T
    wg_t = w_gate.T

    grid = (n // BLOCK,)
    rep = lambda i: (0, 0)
    rep3 = lambda i: (0, 0, 0)
    out = pl.pallas_call(
        _block_body,
        grid=grid,
        in_specs=[
            pl.BlockSpec((BLOCK, l), lambda i: (i, 0)),
            pl.BlockSpec((CHAR_VOCAB, E_CHAR), rep),
            pl.BlockSpec((KSIZE, E_CHAR, EMBED), rep3),
            pl.BlockSpec((EMBED,), lambda i: (0,)),
            pl.BlockSpec((EMBED, EMBED), rep),
            pl.BlockSpec((EMBED,), lambda i: (0,)),
            pl.BlockSpec((EMBED, EMBED), rep),
            pl.BlockSpec((EMBED,), lambda i: (0,)),
        ],
        out_specs=pl.BlockSpec((BLOCK, EMBED), lambda i: (i, 0)),
        out_shape=jax.ShapeDtypeStruct((n, EMBED), jnp.float32),
    )(ids, emb_table, convw_t, conv_b, wp_t, b_proj, wg_t, b_gate)
    return out.reshape(s, b, EMBED)
